# baseline (speedup 1.0000x reference)
import functools

import jax
import jax.numpy as jnp
from jax import lax
from jax.experimental import pallas as pl
from jax.experimental.pallas import tpu as pltpu
from jax.experimental.pallas import tpu_sc as plsc

B, N, R, D, H, REL = 4, 10000, 16, 128, 128, 512
BN = B * N
NC, NS = 2, 16
NW = NC * NS
ROWS_PER_W = BN // NW
CHUNK = 5
EDGES = CHUNK * R
STEPS = ROWS_PER_W // CHUNK
NCHUNKS = BN // CHUNK
SINKB = BN + REL
NSINK = 256
DV = D // 16


_mesh = plsc.VectorSubcoreMesh(
    core_axis_name="c", subcore_axis_name="s", num_cores=NC, num_subcores=NS
)


@functools.partial(
    pl.kernel,
    out_type=jax.ShapeDtypeStruct((NCHUNKS, CHUNK, D), jnp.float32),
    mesh=_mesh,
    scratch_types=[
        pltpu.VMEM((3, EDGES), jnp.int32),
        pltpu.VMEM((3, EDGES), jnp.int32),
        pltpu.VMEM((EDGES,), jnp.int32),
        pltpu.VMEM((EDGES,), jnp.int32),
        pltpu.VMEM((EDGES,), jnp.int32),
        pltpu.VMEM((EDGES,), jnp.int32),
        pltpu.VMEM((EDGES, D), jnp.float32),
        pltpu.VMEM((EDGES, D), jnp.float32),
        pltpu.VMEM((EDGES, D), jnp.float32),
        pltpu.VMEM((EDGES, D), jnp.float32),
        pltpu.VMEM((CHUNK, D), jnp.float32),
        pltpu.VMEM((CHUNK, D), jnp.float32),
        pltpu.VMEM_SHARED((REL + NSINK, D), jnp.float32),
        pltpu.SemaphoreType.DMA,
        pltpu.SemaphoreType.DMA,
        pltpu.SemaphoreType.DMA,
        pltpu.SemaphoreType.DMA,
        pltpu.SemaphoreType.DMA,
        pltpu.SemaphoreType.DMA,
        pltpu.SemaphoreType.DMA,
        pltpu.SemaphoreType.DMA,
    ],
)
def _sc_gather(table, relneg, idx_in, out,
               ib0, ib1, ni0, ni1, ri0, ri1, nr0, nr1, rr0, rr1, st0, st1,
               relsh, si0, si1, sgn0, sgn1, sgr0, sgr1, so0, so1):
    ib = (ib0, ib1)
    ni = (ni0, ni1)
    ri = (ri0, ri1)
    nr = (nr0, nr1)
    rr = (rr0, rr1)
    st = (st0, st1)
    si = (si0, si1)
    sgn = (sgn0, sgn1)
    sgr = (sgr0, sgr1)
    so = (so0, so1)

    w = lax.axis_index("s") * NC + lax.axis_index("c")
    g0 = w * STEPS
    row0 = w * ROWS_PER_W
    batch_base = (row0 // N) * N

    def gidx_and_fire(t, p):
        for i in range(EDGES // 16):
            sl = pl.ds(i * 16, 16)
            o = ib[p][0, sl]
            r = ib[p][1, sl]
            m = ib[p][2, sl]
            keep = m > 0
            ni[p][sl] = jnp.where(keep, o + batch_base,
                                  SINKB + (o & (NSINK - 1)))
            ri[p][sl] = jnp.where(keep, r, REL + (r & (NSINK - 1)))
        pltpu.async_copy(table.at[ni[p]], nr[p], sgn[p])
        pltpu.async_copy(relsh.at[ri[p]], rr[p], sgr[p])

    def load_idx(t, p):
        pltpu.async_copy(idx_in.at[g0 + t], ib[p], si[p])

    def drain(dummy_src, dst, sem):
        pltpu.make_async_copy(dummy_src, dst, sem).wait()

    @pl.when(lax.axis_index("s") == 0)
    def _():
        pltpu.sync_copy(relneg, relsh)
    plsc.subcore_barrier()

    load_idx(0, 0)
    load_idx(1, 1)
    drain(idx_in.at[0], ib[0], si[0])
    gidx_and_fire(0, 0)

    def pair(k, carry):
        for par in (0, 1):
            t = 2 * k + par
            q = 1 - par
            if par == 0:
                drain(idx_in.at[0], ib[q], si[q])
                gidx_and_fire(t + 1, q)
            else:
                @pl.when(k < STEPS // 2 - 1)
                def _():
                    drain(idx_in.at[0], ib[q], si[q])
                    gidx_and_fire(t + 1, q)
            @pl.when(k < STEPS // 2 - 1)
            def _():
                load_idx(t + 2, par)
            drain(table.at[pl.ds(0, EDGES)], nr[par], sgn[par])
            drain(table.at[pl.ds(0, EDGES)], rr[par], sgr[par])
            @pl.when(k >= 1)
            def _():
                drain(st[par], out.at[0], so[par])
            for c in range(CHUNK):
                def ebody(eg, acc):
                    row = c * R + eg * 4
                    out = list(acc)
                    for j in range(4):
                        for d in range(DV):
                            out[d] = (out[d]
                                      + nr[par][row + j, pl.ds(d * 16, 16)]
                                      + rr[par][row + j, pl.ds(d * 16, 16)])
                    return tuple(out)
                acc = lax.fori_loop(
                    0, R // 4, ebody,
                    tuple(jnp.zeros((16,), jnp.float32) for _ in range(DV)),
                )
                for d in range(DV):
                    st[par][c, pl.ds(d * 16, 16)] = acc[d]
            pltpu.async_copy(st[par], out.at[g0 + t], so[par])
        return carry

    lax.fori_loop(0, STEPS // 2, pair, 0)
    drain(st[0], out.at[0], so[0])
    drain(st[1], out.at[0], so[1])


RB = 400


def _tc_body(s_ref, node_ref, mask_ref, w0_ref, w1_ref, out_ref):
    m = mask_ref[...].astype(jnp.float32)
    elen = jnp.maximum(jnp.sum(m, axis=1, keepdims=True), 1.0)
    eh = s_ref[...] * (1.0 / (elen * elen))
    sh = lax.dot_general(node_ref[...], w0_ref[...],
                         (((1,), (1,)), ((), ())),
                         preferred_element_type=jnp.float32)
    eh = lax.dot_general(eh, w1_ref[...],
                         (((1,), (1,)), ((), ())),
                         preferred_element_type=jnp.float32)
    out_ref[...] = jnp.maximum(sh + eh, 0.0)


_tc_finish = pl.pallas_call(
    _tc_body,
    grid=(BN // RB,),
    in_specs=[
        pl.BlockSpec((RB, D), lambda i: (i, 0)),
        pl.BlockSpec((RB, D), lambda i: (i, 0)),
        pl.BlockSpec((RB, R), lambda i: (i, 0)),
        pl.BlockSpec((H, D), lambda i: (0, 0)),
        pl.BlockSpec((H, D), lambda i: (0, 0)),
    ],
    out_specs=pl.BlockSpec((RB, H), lambda i: (i, 0)),
    out_shape=jax.ShapeDtypeStruct((BN, H), jnp.float32),
)


def kernel(node, edge_rel, edge_obj, edge_mask, rel_table, W0, W1):
    node_flat = node.reshape(BN, D)
    table = jnp.concatenate(
        [node_flat, -rel_table, jnp.zeros((NSINK, D), jnp.float32)], axis=0)
    idx_in = jnp.stack(
        [edge_obj.reshape(NCHUNKS, EDGES).astype(jnp.int32),
         edge_rel.reshape(NCHUNKS, EDGES).astype(jnp.int32),
         edge_mask.reshape(NCHUNKS, EDGES).astype(jnp.int32)], axis=1)
    relneg = jnp.concatenate(
        [-rel_table, jnp.zeros((NSINK, D), jnp.float32)], axis=0)
    s = _sc_gather(table, relneg, idx_in).reshape(BN, D)
    out = _tc_finish(s, node_flat, edge_mask.reshape(BN, R).astype(jnp.int32),
                     W0, W1)
    return out.reshape(B, N, H)


# speedup vs baseline: 1.1360x; 1.1360x over previous
import functools

import jax
import jax.numpy as jnp
from jax import lax
from jax.experimental import pallas as pl
from jax.experimental.pallas import tpu as pltpu
from jax.experimental.pallas import tpu_sc as plsc

B, N, R, D, H, REL = 4, 10000, 16, 128, 128, 512
BN = B * N
NC, NS = 2, 16
CHUNK = 5
EDGES = CHUNK * R
NCHUNKS = BN // CHUNK
ROWS_PT = N // NS
STEPS_P = ROWS_PT // CHUNK
GPB = N // CHUNK
NSINK = 256
SINKB = N + REL
DV = D // 16


_mesh = plsc.VectorSubcoreMesh(
    core_axis_name="c", subcore_axis_name="s", num_cores=NC, num_subcores=NS
)


@functools.partial(
    pl.kernel,
    out_type=jax.ShapeDtypeStruct((NCHUNKS, CHUNK, D), jnp.float32),
    mesh=_mesh,
    scratch_types=[
        pltpu.VMEM((3, EDGES), jnp.int32),
        pltpu.VMEM((3, EDGES), jnp.int32),
        pltpu.VMEM((EDGES,), jnp.int32),
        pltpu.VMEM((EDGES,), jnp.int32),
        pltpu.VMEM((EDGES,), jnp.int32),
        pltpu.VMEM((EDGES,), jnp.int32),
        pltpu.VMEM((EDGES, D), jnp.float32),
        pltpu.VMEM((EDGES, D), jnp.float32),
        pltpu.VMEM((EDGES, D), jnp.float32),
        pltpu.VMEM((EDGES, D), jnp.float32),
        pltpu.VMEM((CHUNK, D), jnp.float32),
        pltpu.VMEM((CHUNK, D), jnp.float32),
        pltpu.VMEM_SHARED((N + REL + NSINK, D), jnp.float32),
        pltpu.SemaphoreType.DMA,
        pltpu.SemaphoreType.DMA,
        pltpu.SemaphoreType.DMA,
        pltpu.SemaphoreType.DMA,
        pltpu.SemaphoreType.DMA,
        pltpu.SemaphoreType.DMA,
        pltpu.SemaphoreType.DMA,
        pltpu.SemaphoreType.DMA,
    ],
)
def _sc_gather(node_hbm, relneg, idx_in, out,
               ib0, ib1, ni0, ni1, ri0, ri1, nr0, nr1, rr0, rr1, st0, st1,
               shv, si0, si1, sgn0, sgn1, sgr0, sgr1, so0, so1):
    ib = (ib0, ib1)
    ni = (ni0, ni1)
    ri = (ri0, ri1)
    nr = (nr0, nr1)
    rr = (rr0, rr1)
    st = (st0, st1)
    si = (si0, si1)
    sgn = (sgn0, sgn1)
    sgr = (sgr0, sgr1)
    so = (so0, so1)

    c = lax.axis_index("c")
    s = lax.axis_index("s")

    def drain(dummy_src, dst, sem):
        pltpu.make_async_copy(dummy_src, dst, sem).wait()

    @pl.when(s == 0)
    def _():
        pltpu.sync_copy(relneg, shv.at[pl.ds(N, REL + NSINK)])

    for ph in range(2):
        bt = c * 2 + ph
        pltpu.sync_copy(
            node_hbm.at[pl.ds(bt * N + s * 624, 624)],
            shv.at[pl.ds(s * 624, 624)])
        @pl.when(s == 0)
        def _():
            pltpu.sync_copy(
                node_hbm.at[pl.ds(bt * N + NS * 624, N - NS * 624)],
                shv.at[pl.ds(NS * 624, N - NS * 624)])
        plsc.subcore_barrier()

        g0 = bt * GPB + s * STEPS_P

        def gidx_and_fire(t, p):
            for i in range(EDGES // 16):
                sl = pl.ds(i * 16, 16)
                o = ib[p][0, sl]
                r = ib[p][1, sl]
                m = ib[p][2, sl]
                keep = m > 0
                ni[p][sl] = jnp.where(keep, o, SINKB + (o & (NSINK - 1)))
                ri[p][sl] = jnp.where(keep, N + r,
                                      SINKB + (r & (NSINK - 1)))
            pltpu.async_copy(shv.at[ni[p]], nr[p], sgn[p])
            pltpu.async_copy(shv.at[ri[p]], rr[p], sgr[p])

        def load_idx(t, p):
            pltpu.async_copy(idx_in.at[g0 + t], ib[p], si[p])

        def acc_store(t, par):
            for cc in range(CHUNK):
                def ebody(eg, acc):
                    row = cc * R + eg * 4
                    o2 = list(acc)
                    for j in range(4):
                        for d in range(DV):
                            o2[d] = (o2[d]
                                     + nr[par][row + j, pl.ds(d * 16, 16)]
                                     + rr[par][row + j, pl.ds(d * 16, 16)])
                    return tuple(o2)
                acc = lax.fori_loop(
                    0, R // 4, ebody,
                    tuple(jnp.zeros((16,), jnp.float32) for _ in range(DV)),
                )
                for d in range(DV):
                    st[par][cc, pl.ds(d * 16, 16)] = acc[d]
            pltpu.async_copy(st[par], out.at[g0 + t], so[par])

        load_idx(0, 0)
        load_idx(1, 1)
        drain(idx_in.at[0], ib[0], si[0])
        gidx_and_fire(0, 0)

        def pair(k, carry):
            for par in (0, 1):
                t = 2 * k + par
                q = 1 - par
                drain(idx_in.at[0], ib[q], si[q])
                gidx_and_fire(t + 1, q)
                if par == 0:
                    load_idx(t + 2, par)
                else:
                    @pl.when(k < (STEPS_P - 3) // 2)
                    def _():
                        load_idx(t + 2, par)
                drain(node_hbm.at[pl.ds(0, EDGES)], nr[par], sgn[par])
                drain(node_hbm.at[pl.ds(0, EDGES)], rr[par], sgr[par])
                @pl.when(k >= 1)
                def _():
                    drain(st[par], out.at[0], so[par])
                acc_store(t, par)
            return carry

        lax.fori_loop(0, (STEPS_P - 1) // 2, pair, 0)

        drain(node_hbm.at[pl.ds(0, EDGES)], nr[0], sgn[0])
        drain(node_hbm.at[pl.ds(0, EDGES)], rr[0], sgr[0])
        drain(st[0], out.at[0], so[0])
        acc_store(STEPS_P - 1, 0)

        drain(st[0], out.at[0], so[0])
        drain(st[1], out.at[0], so[1])
        plsc.subcore_barrier()


RB = 400


def _tc_self_body(node_ref, w0_ref, out_ref):
    out_ref[...] = lax.dot_general(node_ref[...], w0_ref[...],
                                   (((1,), (1,)), ((), ())),
                                   preferred_element_type=jnp.float32)


_tc_self = pl.pallas_call(
    _tc_self_body,
    grid=(BN // RB,),
    in_specs=[
        pl.BlockSpec((RB, D), lambda i: (i, 0)),
        pl.BlockSpec((H, D), lambda i: (0, 0)),
    ],
    out_specs=pl.BlockSpec((RB, H), lambda i: (i, 0)),
    out_shape=jax.ShapeDtypeStruct((BN, H), jnp.float32),
)


def _tc_body(s_ref, sh_ref, mask_ref, w1_ref, out_ref):
    m = mask_ref[...].astype(jnp.float32)
    elen = jnp.maximum(jnp.sum(m, axis=1, keepdims=True), 1.0)
    eh = s_ref[...] * (1.0 / (elen * elen))
    eh = lax.dot_general(eh, w1_ref[...],
                         (((1,), (1,)), ((), ())),
                         preferred_element_type=jnp.float32)
    out_ref[...] = jnp.maximum(sh_ref[...] + eh, 0.0)


_tc_finish = pl.pallas_call(
    _tc_body,
    grid=(BN // RB,),
    in_specs=[
        pl.BlockSpec((RB, D), lambda i: (i, 0)),
        pl.BlockSpec((RB, H), lambda i: (i, 0)),
        pl.BlockSpec((RB, R), lambda i: (i, 0)),
        pl.BlockSpec((H, D), lambda i: (0, 0)),
    ],
    out_specs=pl.BlockSpec((RB, H), lambda i: (i, 0)),
    out_shape=jax.ShapeDtypeStruct((BN, H), jnp.float32),
)


def kernel(node, edge_rel, edge_obj, edge_mask, rel_table, W0, W1):
    node_flat = node.reshape(BN, D)
    relneg = jnp.concatenate(
        [-rel_table, jnp.zeros((NSINK, D), jnp.float32)], axis=0)
    idx_in = jnp.stack(
        [edge_obj.reshape(NCHUNKS, EDGES).astype(jnp.int32),
         edge_rel.reshape(NCHUNKS, EDGES).astype(jnp.int32),
         edge_mask.reshape(NCHUNKS, EDGES).astype(jnp.int32)], axis=1)
    s = _sc_gather(node_flat, relneg, idx_in).reshape(BN, D)
    sh = _tc_self(node_flat, W0)
    out = _tc_finish(s, sh, edge_mask.reshape(BN, R).astype(jnp.int32), W1)
    return out.reshape(B, N, H)
